# baseline (speedup 1.0000x reference)
import functools
import math

import jax
import jax.numpy as jnp
from jax import lax
from jax.experimental import pallas as pl
from jax.experimental.pallas import tpu as pltpu
from jax.experimental.pallas import tpu_sc as plsc

_LABEL_SMOOTHING = 0.1
_NUM_GOALS = 256
_NUM_BUCKETS = 128
_BATCH = 1024
_CONF = 1.0 - _LABEL_SMOOTHING
_SMOOTH = _LABEL_SMOOTHING / _NUM_BUCKETS
_RATIO = _CONF / _SMOOTH
_C_LOG = _BATCH * _NUM_GOALS * (
    (_NUM_BUCKETS - 1) * _SMOOTH * math.log(_SMOOTH) + _CONF * math.log(_CONF)
)

_B_SC = 128
_B_TC = _BATCH - _B_SC
_BB = 128

_NW = 32
_NB_W = _B_SC // _NW
_KH = _NUM_BUCKETS // 2
_CHW = _KH * _NUM_GOALS
_ROW_W = _NUM_BUCKETS * _NUM_GOALS
_NCHUNK = _NB_W * 2


def _tc_kernel(tgt_ref, out_blk_ref, acc_ref, col_ref):
    i = pl.program_id(0)
    x = out_blk_ref[...]
    tgt = tgt_ref[...]
    bucket = lax.broadcasted_iota(jnp.int32, x.shape, 1)
    z = jnp.where(bucket == tgt[:, None, :], x * _RATIO, x)
    z2 = z.reshape(_BB * _NUM_BUCKETS, _NUM_GOALS)
    ones = jnp.ones((8, _BB * _NUM_BUCKETS), jnp.float32)
    col = jax.lax.dot_general(
        ones, z2, (((1,), (0,)), ((), ())),
        precision=lax.Precision.DEFAULT,
        preferred_element_type=jnp.float32,
    )

    @pl.when(i == 0)
    def _init():
        col_ref[...] = jnp.zeros_like(col_ref)

    col_ref[...] += col

    @pl.when(i == pl.num_programs(0) - 1)
    def _fini():
        acc_ref[0, 0] = jnp.float32(_C_LOG) - _SMOOTH * jnp.sum(
            col_ref[0:1, :]
        )


def _sc_kernel(x_hbm, tgt_hbm, out_hbm, tvm, buf0, buf1, outv, sem0, sem1):
    wid = lax.axis_index("s") * 2 + lax.axis_index("c")
    base = _B_TC + wid * _NB_W
    pltpu.sync_copy(tgt_hbm.at[pl.ds(base, _NB_W)], tvm)

    bufs = (buf0, buf1)
    sems = (sem0, sem1)

    def issue(t):
        rb, half = divmod(t, 2)
        return pltpu.async_copy(
            x_hbm.at[base + rb, pl.ds(half * _KH, _KH), :],
            bufs[t % 2],
            sems[t % 2],
        )

    cp = issue(0)
    zero = jnp.zeros((16,), jnp.float32)
    iota16 = lax.iota(jnp.int32, 16)
    accs = (zero,) * 8
    ga = zero
    for t in range(_NCHUNK):
        rb, half = divmod(t, 2)
        k0 = half * _KH
        cp.wait()
        if t + 1 < _NCHUNK:
            cp = issue(t + 1)
        buf = bufs[t % 2]

        def _inner(i, acc, buf=buf):
            out = list(acc)
            for s in range(8):
                c = s * 32
                out[s] = out[s] + buf[i, pl.ds(c, 16)]
                out[s] = out[s] + buf[i, pl.ds(c + 16, 16)]
            return tuple(out)

        accs = plsc.parallel_loop(0, _KH, 1, unroll=4, carry=accs)(_inner)

        for j in range(0, _NUM_GOALS, 16):
            t16 = tvm[rb, pl.ds(j, 16)]
            m = (t16 >= k0) & (t16 < k0 + _KH)
            kr = jnp.where(m, t16 - k0, 0)
            vals = plsc.load_gather(buf, [kr, iota16 + j])
            ga = ga + jnp.where(m, vals, zero)

    dense = accs[0]
    for s in range(1, 8):
        dense = dense + accs[s]
    outv[...] = _SMOOTH * dense + (_CONF - _SMOOTH) * ga
    pltpu.sync_copy(outv, out_hbm.at[wid])


@functools.partial(
    pl.kernel,
    out_type=jax.ShapeDtypeStruct((_NW, 16), jnp.float32),
    mesh=plsc.VectorSubcoreMesh(core_axis_name="c", subcore_axis_name="s"),
    compiler_params=pltpu.CompilerParams(
        needs_layout_passes=False, vmem_limit_bytes=2 * 1024 * 1024
    ),
    scratch_types=[
        pltpu.VMEM((_NB_W, _NUM_GOALS), jnp.int32),
        pltpu.VMEM((_KH, _NUM_GOALS), jnp.float32),
        pltpu.VMEM((_KH, _NUM_GOALS), jnp.float32),
        pltpu.VMEM((16,), jnp.float32),
        pltpu.SemaphoreType.DMA,
        pltpu.SemaphoreType.DMA,
    ],
)
def _sc_call(x_hbm, tgt_hbm, out_hbm, tvm, buf0, buf1, outv, sem0, sem1):
    _sc_kernel(x_hbm, tgt_hbm, out_hbm, tvm, buf0, buf1, outv, sem0, sem1)


def kernel(output, target, one_hot):
    del one_hot
    grid = _B_TC // _BB
    acc = pl.pallas_call(
        _tc_kernel,
        grid=(grid,),
        in_specs=[
            pl.BlockSpec((_BB, _NUM_GOALS), lambda i: (i, 0)),
            pl.BlockSpec((_BB, _NUM_BUCKETS, _NUM_GOALS), lambda i: (i, 0, 0)),
        ],
        out_specs=pl.BlockSpec(
            (1, 1), lambda i: (0, 0), memory_space=pltpu.SMEM
        ),
        out_shape=jax.ShapeDtypeStruct((1, 1), jnp.float32),
        scratch_shapes=[pltpu.VMEM((8, _NUM_GOALS), jnp.float32)],
        compiler_params=pltpu.CompilerParams(
            vmem_limit_bytes=48 * 1024 * 1024
        ),
    )(target, output)
    sc_part = _sc_call(output, target)
    return acc[0, 0] - jnp.sum(sc_part)


# speedup vs baseline: 1.0005x; 1.0005x over previous
import functools
import math

import jax
import jax.numpy as jnp
from jax import lax
from jax.experimental import pallas as pl
from jax.experimental.pallas import tpu as pltpu
from jax.experimental.pallas import tpu_sc as plsc

_LABEL_SMOOTHING = 0.1
_NUM_GOALS = 256
_NUM_BUCKETS = 128
_BATCH = 1024
_CONF = 1.0 - _LABEL_SMOOTHING
_SMOOTH = _LABEL_SMOOTHING / _NUM_BUCKETS
_RATIO = _CONF / _SMOOTH
_C_LOG = _BATCH * _NUM_GOALS * (
    (_NUM_BUCKETS - 1) * _SMOOTH * math.log(_SMOOTH) + _CONF * math.log(_CONF)
)

_B_SC = 128
_B_TC = _BATCH - _B_SC
_BB = 128

_NW = 32
_NB_W = _B_SC // _NW
_KH = _NUM_BUCKETS // 2
_CHW = _KH * _NUM_GOALS
_ROW_W = _NUM_BUCKETS * _NUM_GOALS
_NCHUNK = _NB_W * 2


def _tc_kernel(tgt_ref, out_blk_ref, acc_ref, col_ref):
    i = pl.program_id(0)
    x = out_blk_ref[...]
    tgt = tgt_ref[...]
    bucket = lax.broadcasted_iota(jnp.int32, x.shape, 1)
    z = jnp.where(bucket == tgt[:, None, :], x * _RATIO, x)
    z2 = z.reshape(_BB * _NUM_BUCKETS, _NUM_GOALS)
    ones = jnp.ones((8, _BB * _NUM_BUCKETS), jnp.float32)
    col = jax.lax.dot_general(
        ones, z2, (((1,), (0,)), ((), ())),
        precision=lax.Precision.DEFAULT,
        preferred_element_type=jnp.float32,
    )

    @pl.when(i == 0)
    def _init():
        col_ref[...] = jnp.zeros_like(col_ref)

    col_ref[...] += col

    @pl.when(i == pl.num_programs(0) - 1)
    def _fini():
        acc_ref[0, 0] = jnp.float32(_C_LOG) - _SMOOTH * jnp.sum(
            col_ref[0:1, :]
        )


def _sc_kernel(x_hbm, tgt_hbm, out_hbm, tvm, buf0, buf1, outv, sem0, sem1):
    wid = lax.axis_index("s") * 2 + lax.axis_index("c")
    base = _B_TC + wid * _NB_W
    pltpu.sync_copy(tgt_hbm.at[pl.ds(base, _NB_W)], tvm)

    bufs = (buf0, buf1)
    sems = (sem0, sem1)

    def issue(t):
        rb, half = divmod(t, 2)
        return pltpu.async_copy(
            x_hbm.at[base + rb, pl.ds(half * _KH, _KH), :],
            bufs[t % 2],
            sems[t % 2],
        )

    cp = issue(0)
    zero = jnp.zeros((16,), jnp.float32)
    iota16 = lax.iota(jnp.int32, 16)
    accs = (zero,) * 8
    ga = zero
    for t in range(_NCHUNK):
        rb, half = divmod(t, 2)
        k0 = half * _KH
        cp.wait()
        if t + 1 < _NCHUNK:
            cp = issue(t + 1)
        buf = bufs[t % 2]

        def _inner(i, acc, buf=buf):
            out = list(acc)
            for s in range(8):
                c = s * 32
                out[s] = out[s] + buf[i, pl.ds(c, 16)]
                out[s] = out[s] + buf[i, pl.ds(c + 16, 16)]
            return tuple(out)

        accs = plsc.parallel_loop(0, _KH, 1, unroll=4, carry=accs)(_inner)

        for j in range(0, _NUM_GOALS, 16):
            t16 = tvm[rb, pl.ds(j, 16)]
            m = (t16 >= k0) & (t16 < k0 + _KH)
            kr = jnp.where(m, t16 - k0, 0)
            vals = plsc.load_gather(buf, [kr, iota16 + j])
            ga = ga + jnp.where(m, vals, zero)

    dense = accs[0]
    for s in range(1, 8):
        dense = dense + accs[s]
    outv[...] = _SMOOTH * dense + (_CONF - _SMOOTH) * ga
    pltpu.sync_copy(outv, out_hbm.at[wid])


@functools.partial(
    pl.kernel,
    out_type=jax.ShapeDtypeStruct((_NW, 16), jnp.float32),
    mesh=plsc.VectorSubcoreMesh(core_axis_name="c", subcore_axis_name="s"),
    compiler_params=pltpu.CompilerParams(
        needs_layout_passes=False,
        vmem_limit_bytes=2 * 1024 * 1024,
        skip_device_barrier=True,
    ),
    scratch_types=[
        pltpu.VMEM((_NB_W, _NUM_GOALS), jnp.int32),
        pltpu.VMEM((_KH, _NUM_GOALS), jnp.float32),
        pltpu.VMEM((_KH, _NUM_GOALS), jnp.float32),
        pltpu.VMEM((16,), jnp.float32),
        pltpu.SemaphoreType.DMA,
        pltpu.SemaphoreType.DMA,
    ],
)
def _sc_call(x_hbm, tgt_hbm, out_hbm, tvm, buf0, buf1, outv, sem0, sem1):
    _sc_kernel(x_hbm, tgt_hbm, out_hbm, tvm, buf0, buf1, outv, sem0, sem1)


def kernel(output, target, one_hot):
    del one_hot
    grid = _B_TC // _BB
    acc = pl.pallas_call(
        _tc_kernel,
        grid=(grid,),
        in_specs=[
            pl.BlockSpec((_BB, _NUM_GOALS), lambda i: (i, 0)),
            pl.BlockSpec((_BB, _NUM_BUCKETS, _NUM_GOALS), lambda i: (i, 0, 0)),
        ],
        out_specs=pl.BlockSpec(
            (1, 1), lambda i: (0, 0), memory_space=pltpu.SMEM
        ),
        out_shape=jax.ShapeDtypeStruct((1, 1), jnp.float32),
        scratch_shapes=[pltpu.VMEM((8, _NUM_GOALS), jnp.float32)],
        compiler_params=pltpu.CompilerParams(
            vmem_limit_bytes=48 * 1024 * 1024
        ),
    )(target, output)
    sc_part = _sc_call(output, target)
    return acc[0, 0] - jnp.sum(sc_part)


# speedup vs baseline: 1.4449x; 1.4441x over previous
import math

import jax
import jax.numpy as jnp
from jax import lax
from jax.experimental import pallas as pl
from jax.experimental.pallas import tpu as pltpu

_LABEL_SMOOTHING = 0.1
_NUM_GOALS = 256
_NUM_BUCKETS = 128
_BATCH = 1024
_CONF = 1.0 - _LABEL_SMOOTHING
_SMOOTH = _LABEL_SMOOTHING / _NUM_BUCKETS
_RATIO = _CONF / _SMOOTH
_C_LOG = _BATCH * _NUM_GOALS * (
    (_NUM_BUCKETS - 1) * _SMOOTH * math.log(_SMOOTH) + _CONF * math.log(_CONF)
)

_BB = 128


def _loss_kernel(tgt_ref, out_blk_ref, acc_ref, col_ref):
    i = pl.program_id(0)
    x = out_blk_ref[...]
    tgt = tgt_ref[...]
    bucket = lax.broadcasted_iota(jnp.int32, x.shape, 1)
    z = jnp.where(bucket == tgt[:, None, :], x * _RATIO, x)
    z2 = z.reshape(_BB * _NUM_BUCKETS, _NUM_GOALS)
    ones = jnp.ones((8, _BB * _NUM_BUCKETS), jnp.float32)
    col = jax.lax.dot_general(
        ones, z2, (((1,), (0,)), ((), ())),
        precision=lax.Precision.DEFAULT,
        preferred_element_type=jnp.float32,
    )

    @pl.when(i == 0)
    def _init():
        col_ref[...] = jnp.zeros_like(col_ref)

    col_ref[...] += col

    @pl.when(i == pl.num_programs(0) - 1)
    def _fini():
        acc_ref[0, 0] = jnp.float32(_C_LOG) - _SMOOTH * jnp.sum(
            col_ref[0:1, :]
        )


def kernel(output, target, one_hot):
    del one_hot
    grid = _BATCH // _BB
    acc = pl.pallas_call(
        _loss_kernel,
        grid=(grid,),
        in_specs=[
            pl.BlockSpec((_BB, _NUM_GOALS), lambda i: (i, 0)),
            pl.BlockSpec((_BB, _NUM_BUCKETS, _NUM_GOALS), lambda i: (i, 0, 0)),
        ],
        out_specs=pl.BlockSpec(
            (1, 1), lambda i: (0, 0), memory_space=pltpu.SMEM
        ),
        out_shape=jax.ShapeDtypeStruct((1, 1), jnp.float32),
        scratch_shapes=[pltpu.VMEM((8, _NUM_GOALS), jnp.float32)],
    )(target, output)
    return acc[0, 0]


# speedup vs baseline: 1.5055x; 1.0420x over previous
import math

import jax
import jax.numpy as jnp
from jax import lax
from jax.experimental import pallas as pl
from jax.experimental.pallas import tpu as pltpu

_LABEL_SMOOTHING = 0.1
_NUM_GOALS = 256
_NUM_BUCKETS = 128
_BATCH = 1024
_CONF = 1.0 - _LABEL_SMOOTHING
_SMOOTH = _LABEL_SMOOTHING / _NUM_BUCKETS
_RATIO = _CONF / _SMOOTH
_C_LOG = _BATCH * _NUM_GOALS * (
    (_NUM_BUCKETS - 1) * _SMOOTH * math.log(_SMOOTH) + _CONF * math.log(_CONF)
)

_BB = 64


def _loss_kernel(tgt_ref, out_blk_ref, acc_ref, col_ref):
    i = pl.program_id(0)
    x = out_blk_ref[...]
    tgt = tgt_ref[...]
    bucket = lax.broadcasted_iota(jnp.int32, x.shape, 1)
    z = jnp.where(bucket == tgt[:, None, :], x * _RATIO, x)
    z2 = z.reshape(_BB * _NUM_BUCKETS, _NUM_GOALS)
    ones = jnp.ones((8, _BB * _NUM_BUCKETS), jnp.float32)
    col = jax.lax.dot_general(
        ones, z2, (((1,), (0,)), ((), ())),
        precision=lax.Precision.DEFAULT,
        preferred_element_type=jnp.float32,
    )

    @pl.when(i == 0)
    def _init():
        col_ref[...] = jnp.zeros_like(col_ref)

    col_ref[...] += col

    @pl.when(i == pl.num_programs(0) - 1)
    def _fini():
        acc_ref[0, 0] = jnp.float32(_C_LOG) - _SMOOTH * jnp.sum(
            col_ref[0:1, :]
        )


def kernel(output, target, one_hot):
    del one_hot
    grid = _BATCH // _BB
    acc = pl.pallas_call(
        _loss_kernel,
        grid=(grid,),
        in_specs=[
            pl.BlockSpec((_BB, _NUM_GOALS), lambda i: (i, 0)),
            pl.BlockSpec((_BB, _NUM_BUCKETS, _NUM_GOALS), lambda i: (i, 0, 0)),
        ],
        out_specs=pl.BlockSpec(
            (1, 1), lambda i: (0, 0), memory_space=pltpu.SMEM
        ),
        out_shape=jax.ShapeDtypeStruct((1, 1), jnp.float32),
        scratch_shapes=[pltpu.VMEM((8, _NUM_GOALS), jnp.float32)],
    )(target, output)
    return acc[0, 0]
